# Initial kernel scaffold; baseline (speedup 1.0000x reference)
import jax
import jax.numpy as jnp
from jax.experimental import pallas as pl


def kernel(x):
    raise NotImplementedError("write your pallas kernel here")


# speedup vs baseline: 3.1080x; 3.1080x over previous
import jax
import jax.numpy as jnp
from jax.experimental import pallas as pl

NF = 26
NP = (NF * (NF - 1)) // 2
BBLK = 256


def _body(x_ref, o_ref):
    x = x_ref[...]
    parts = []
    for i in range(NF - 1):
        xi = x[:, i : i + 1, :]
        rest = x[:, i + 1 :, :]
        parts.append(jnp.sum(xi * rest, axis=-1))
    o_ref[...] = jnp.concatenate(parts, axis=1)


def kernel(x):
    B = x.shape[0]
    grid = (B // BBLK,)
    return pl.pallas_call(
        _body,
        grid=grid,
        in_specs=[pl.BlockSpec((BBLK, NF, 128), lambda b: (b, 0, 0))],
        out_specs=pl.BlockSpec((BBLK, NP), lambda b: (b, 0)),
        out_shape=jax.ShapeDtypeStruct((B, NP), jnp.float32),
    )(x)
